# baseline (speedup 1.0000x reference)
import jax
import jax.numpy as jnp
from jax.experimental import pallas as pl
from jax.experimental.pallas import tpu as pltpu

_B, _S, _D = 16, 1024, 256
_ROWS = _B * _S
_NDMA = 4
_SLICE = _ROWS // _NDMA


def _dma_kernel(x_hbm, o_hbm, sems):
    for i in range(_NDMA):
        pltpu.make_async_copy(
            x_hbm.at[pl.ds(i * _SLICE, _SLICE), :],
            o_hbm.at[pl.ds(i * _SLICE, _SLICE), :],
            sems.at[i],
        ).start()
    for i in range(_NDMA):
        pltpu.make_async_copy(
            x_hbm.at[pl.ds(i * _SLICE, _SLICE), :],
            o_hbm.at[pl.ds(i * _SLICE, _SLICE), :],
            sems.at[i],
        ).wait()


def kernel(x, W):
    del W
    flat = x.reshape(_ROWS, _D)
    out = pl.pallas_call(
        _dma_kernel,
        in_specs=[pl.BlockSpec(memory_space=pltpu.MemorySpace.HBM)],
        out_specs=pl.BlockSpec(memory_space=pltpu.MemorySpace.HBM),
        out_shape=jax.ShapeDtypeStruct((_ROWS, _D), x.dtype),
        scratch_shapes=[pltpu.SemaphoreType.DMA((_NDMA,))],
    )(flat)
    return out.reshape(x.shape)


# speedup vs baseline: 9.6847x; 9.6847x over previous
import jax
import jax.numpy as jnp
from jax.experimental import pallas as pl
from jax.experimental.pallas import tpu as pltpu

_B, _S, _D = 16, 1024, 256
_R, _C = 512, 8192
_BLKR = 128


def _copy_kernel(x_ref, o_ref):
    o_ref[...] = x_ref[...]


def kernel(x, W):
    del W
    flat = x.reshape(_R, _C)
    out = pl.pallas_call(
        _copy_kernel,
        grid=(_R // _BLKR,),
        in_specs=[pl.BlockSpec((_BLKR, _C), lambda i: (i, 0))],
        out_specs=pl.BlockSpec((_BLKR, _C), lambda i: (i, 0)),
        out_shape=jax.ShapeDtypeStruct((_R, _C), x.dtype),
        compiler_params=pltpu.CompilerParams(
            dimension_semantics=("parallel",),
        ),
    )(flat)
    return out.reshape(x.shape)


# speedup vs baseline: 29.5400x; 3.0502x over previous
import jax
import jax.numpy as jnp
from jax.experimental import pallas as pl
from jax.experimental.pallas import tpu as pltpu

_B, _S, _D = 16, 1024, 256
_ROWS = _B * _S
_BLK = 1024


def _copy_kernel(x_ref, o_ref):
    o_ref[...] = x_ref[...]


def kernel(x, W):
    del W
    flat = x.reshape(_ROWS, _D)
    out = pl.pallas_call(
        _copy_kernel,
        grid=(_ROWS // _BLK,),
        in_specs=[pl.BlockSpec((_BLK, _D), lambda i: (i, 0))],
        out_specs=pl.BlockSpec((_BLK, _D), lambda i: (i, 0)),
        out_shape=jax.ShapeDtypeStruct((_ROWS, _D), x.dtype),
        compiler_params=pltpu.CompilerParams(
            dimension_semantics=("parallel",),
        ),
    )(flat)
    return out.reshape(x.shape)


# speedup vs baseline: 41.9370x; 1.4197x over previous
import jax
import jax.numpy as jnp
from jax.experimental import pallas as pl
from jax.experimental.pallas import tpu as pltpu

_B, _S, _D = 16, 1024, 256
_ROWS = _B * _S
_BLK = 4096


def _copy_kernel(x_ref, o_ref):
    o_ref[...] = x_ref[...]


def kernel(x, W):
    del W
    flat = x.reshape(_ROWS, _D)
    out = pl.pallas_call(
        _copy_kernel,
        grid=(_ROWS // _BLK,),
        in_specs=[pl.BlockSpec((_BLK, _D), lambda i: (i, 0))],
        out_specs=pl.BlockSpec((_BLK, _D), lambda i: (i, 0)),
        out_shape=jax.ShapeDtypeStruct((_ROWS, _D), x.dtype),
        compiler_params=pltpu.CompilerParams(
            dimension_semantics=("parallel",),
        ),
    )(flat)
    return out.reshape(x.shape)


# speedup vs baseline: 47.8003x; 1.1398x over previous
import jax
import jax.numpy as jnp
from jax.experimental import pallas as pl
from jax.experimental.pallas import tpu as pltpu

_B, _S, _D = 16, 1024, 256
_ROWS = _B * _S
_BLK = 8192


def _copy_kernel(x_ref, o_ref):
    o_ref[...] = x_ref[...]


def kernel(x, W):
    del W
    flat = x.reshape(_ROWS, _D)
    out = pl.pallas_call(
        _copy_kernel,
        grid=(_ROWS // _BLK,),
        in_specs=[pl.BlockSpec((_BLK, _D), lambda i: (i, 0))],
        out_specs=pl.BlockSpec((_BLK, _D), lambda i: (i, 0)),
        out_shape=jax.ShapeDtypeStruct((_ROWS, _D), x.dtype),
        compiler_params=pltpu.CompilerParams(
            dimension_semantics=("parallel",),
        ),
    )(flat)
    return out.reshape(x.shape)
